# Initial kernel scaffold; baseline (speedup 1.0000x reference)
import jax
import jax.numpy as jnp
from jax.experimental import pallas as pl


def kernel(x, edge_index):
    raise NotImplementedError("write your pallas kernel here")


# speedup vs baseline: 4.0336x; 4.0336x over previous
import functools

import jax
import jax.numpy as jnp
from jax import lax
from jax.experimental import pallas as pl
from jax.experimental.pallas import tpu as pltpu
from jax.experimental.pallas import tpu_sc as plsc

_NUM_HOPS = 3
_SIGMA = 0.1
_N = 10000
_D = 128
_E = 320000

_NC = 2
_NS = 16
_NW = _NC * _NS
_EPW = _E // _NW
_C = 80
_NCHUNK = _EPW // _C
_NP = 10240
_RPS = _NP // _NS
_RC = 128


def _sc_hop_body(h_hbm, src_hbm, dst_hbm, out_hbm,
                 aggr_sh, sidx, didx, srows, drows, obuf, tbuf, sem):
    c = lax.axis_index("c")
    s = lax.axis_index("s")
    wid = c * _NS + s

    z16 = jnp.zeros((16,), jnp.float32)

    def zero_row(i, _):
        for g in range(_D // 16):
            obuf[i, pl.ds(g * 16, 16)] = z16
        return 0

    lax.fori_loop(0, _RC, zero_row, 0)
    for j in range(_RPS // _RC):
        pltpu.sync_copy(obuf, aggr_sh.at[pl.ds(s * _RPS + j * _RC, _RC)])
    plsc.subcore_barrier()

    lanes = lax.iota(jnp.int32, 16)

    def chunk_body(ci, _):
        ebase = wid * _EPW + ci * _C
        pltpu.sync_copy(src_hbm.at[pl.ds(ebase, _C)], sidx)
        pltpu.sync_copy(dst_hbm.at[pl.ds(ebase, _C)], didx)
        cp_s = pltpu.async_copy(h_hbm.at[sidx], srows, sem)
        cp_d = pltpu.async_copy(h_hbm.at[didx], drows, sem)
        cp_s.wait()
        cp_d.wait()

        def group_body(g, _):
            for e in range(16):
                row = g * 16 + e
                acc = srows[row, pl.ds(0, 16)] * drows[row, pl.ds(0, 16)]
                for k in range(1, _D // 16):
                    sl = pl.ds(k * 16, 16)
                    acc = acc + srows[row, sl] * drows[row, sl]
                tbuf[pl.ds(e * 17, 16)] = acc
            tot = plsc.load_gather(tbuf, [lanes * 17])
            for col in range(1, 16):
                tot = tot + plsc.load_gather(tbuf, [lanes * 17 + col])
            alpha = 1.0 / (1.0 + jnp.exp(-tot))
            for e in range(16):
                row = g * 16 + e
                a = alpha[e]
                for k in range(_D // 16):
                    sl = pl.ds(k * 16, 16)
                    srows[row, sl] = srows[row, sl] * a
            return 0

        lax.fori_loop(0, _C // 16, group_body, 0)
        pltpu.sync_copy(srows, aggr_sh.at[didx], add=True)
        return 0

    lax.fori_loop(0, _NCHUNK, chunk_body, 0)
    plsc.subcore_barrier()

    for j in range(_RPS // _RC):
        rb = s * _RPS + j * _RC
        pltpu.sync_copy(aggr_sh.at[pl.ds(rb, _RC)], obuf)
        pltpu.sync_copy(obuf, out_hbm.at[c, pl.ds(rb, _RC)])


_sc_hop = functools.partial(
    pl.kernel,
    out_type=jax.ShapeDtypeStruct((_NC, _NP, _D), jnp.float32),
    mesh=plsc.VectorSubcoreMesh(
        core_axis_name="c", subcore_axis_name="s",
        num_cores=_NC, num_subcores=_NS),
    compiler_params=pltpu.CompilerParams(needs_layout_passes=False),
    scratch_types=[
        pltpu.VMEM_SHARED((_NP, _D), jnp.float32),
        pltpu.VMEM((_C,), jnp.int32),
        pltpu.VMEM((_C,), jnp.int32),
        pltpu.VMEM((_C, _D), jnp.float32),
        pltpu.VMEM((_C, _D), jnp.float32),
        pltpu.VMEM((_RC, _D), jnp.float32),
        pltpu.VMEM((16 * 17,), jnp.float32),
        pltpu.SemaphoreType.DMA,
    ],
)(_sc_hop_body)


def _normalize_rows(y):
    ss = jnp.sum(y * y, axis=1, keepdims=True)
    return y / jnp.maximum(jnp.sqrt(ss), 1e-12)


def _tc_normalize_body(x_ref, o_ref):
    o_ref[...] = _normalize_rows(x_ref[...])


def _tc_combine_body(p0_ref, p1_ref, nz_ref, o_ref):
    o_ref[...] = _normalize_rows(p0_ref[...] + p1_ref[...] + nz_ref[...])


_TC_BLK = 1000

_tc_normalize = pl.pallas_call(
    _tc_normalize_body,
    grid=(_N // _TC_BLK,),
    in_specs=[pl.BlockSpec((_TC_BLK, _D), lambda i: (i, 0))],
    out_specs=pl.BlockSpec((_TC_BLK, _D), lambda i: (i, 0)),
    out_shape=jax.ShapeDtypeStruct((_N, _D), jnp.float32),
)

_tc_combine = pl.pallas_call(
    _tc_combine_body,
    grid=(_N // _TC_BLK,),
    in_specs=[pl.BlockSpec((_TC_BLK, _D), lambda i: (i, 0))] * 3,
    out_specs=pl.BlockSpec((_TC_BLK, _D), lambda i: (i, 0)),
    out_shape=jax.ShapeDtypeStruct((_N, _D), jnp.float32),
)


def kernel(x, edge_index):
    src = edge_index[0]
    dst = edge_index[1]
    h = _tc_normalize(x)
    outs = [h]
    for k in range(_NUM_HOPS):
        noise = _SIGMA * jax.random.normal(
            jax.random.fold_in(jax.random.key(1), k), (_N, _D),
            dtype=jnp.float32)
        parts = _sc_hop(h, src, dst)
        h = _tc_combine(parts[0, :_N], parts[1, :_N], noise)
        outs.append(h)
    return jnp.stack(outs)
